# baseline (speedup 1.0000x reference)
import functools

import jax
import jax.numpy as jnp
from jax import lax
from jax.experimental import pallas as pl
from jax.experimental.pallas import tpu as pltpu
from jax.experimental.pallas import tpu_sc as plsc

MAXLEN = 200
EMBED = 64
BATCH = 1024

_INFO = plsc.get_sparse_core_info()
NC = _INFO.num_cores
NS = _INFO.num_subcores
NW = NC * NS

ROWS = BATCH * MAXLEN
ROWS_PER_W = ROWS // NW
CHUNK = 128
NCHUNK = ROWS_PER_W // CHUNK


def _sc_body(tbl_hbm, idx_hbm, pos_hbm, out_hbm, idx_v, pos_v, buf, sem):
    wid = lax.axis_index("s") * NC + lax.axis_index("c")
    base_row = wid * ROWS_PER_W

    pltpu.sync_copy(idx_hbm.at[wid], idx_v)
    pltpu.sync_copy(pos_hbm, pos_v)

    def chunk_body(c, carry):
        pltpu.async_copy(tbl_hbm.at[idx_v.at[c]], buf, sem).wait()
        base_mod = lax.rem(c * CHUNK, MAXLEN)

        def row_body(j, carry2):
            p = base_mod + j
            for dd in range(EMBED // 16):
                sl = pl.ds(dd * 16, 16)
                plsc.addupdate(buf.at[j, sl], pos_v[p, sl])
            return carry2

        lax.fori_loop(0, CHUNK, row_body, 0, unroll=2)
        pltpu.sync_copy(buf, out_hbm.at[pl.ds(base_row + c * CHUNK, CHUNK)])
        return carry

    lax.fori_loop(0, NCHUNK, chunk_body, 0)


@functools.partial(jax.jit, static_argnames=())
def kernel(x, token_table, pos_table):
    idx = x.reshape(NW, NCHUNK, CHUNK).astype(jnp.int32)
    pos_dup = jnp.concatenate([pos_table, pos_table], axis=0)

    run = pl.kernel(
        _sc_body,
        out_type=jax.ShapeDtypeStruct((ROWS, EMBED), jnp.float32),
        mesh=plsc.VectorSubcoreMesh(core_axis_name="c", subcore_axis_name="s"),
        compiler_params=pltpu.CompilerParams(use_tc_tiling_on_sc=False),
        scratch_types=[
            pltpu.VMEM((NCHUNK, CHUNK), jnp.int32),
            pltpu.VMEM((2 * MAXLEN, EMBED), jnp.float32),
            pltpu.VMEM((CHUNK, EMBED), jnp.float32),
            pltpu.SemaphoreType.DMA,
        ],
    )
    out = run(token_table, idx, pos_dup)
    return out.reshape(BATCH, MAXLEN, EMBED)


# speedup vs baseline: 1.1078x; 1.1078x over previous
import functools

import jax
import jax.numpy as jnp
from jax import lax
from jax.experimental import pallas as pl
from jax.experimental.pallas import tpu as pltpu
from jax.experimental.pallas import tpu_sc as plsc

MAXLEN = 200
EMBED = 64
BATCH = 1024

_INFO = plsc.get_sparse_core_info()
NC = _INFO.num_cores
NS = _INFO.num_subcores
NW = NC * NS
SEQ_PER_W = BATCH // NW
G0 = 128
G1 = MAXLEN - G0


def _sc_body(tbl_hbm, x_hbm, pos_hbm, out_hbm, idx_v, pos_v, buf, sem):
    wid = lax.axis_index("s") * NC + lax.axis_index("c")
    seq_base = wid * SEQ_PER_W

    pltpu.sync_copy(x_hbm.at[pl.ds(seq_base, SEQ_PER_W)], idx_v)
    pltpu.sync_copy(pos_hbm, pos_v)

    def seq_body(s, carry):
        c0 = pltpu.async_copy(
            tbl_hbm.at[idx_v.at[s, pl.ds(0, G0)]], buf.at[pl.ds(0, G0)], sem)
        c1 = pltpu.async_copy(
            tbl_hbm.at[idx_v.at[s, pl.ds(G0, G1)]], buf.at[pl.ds(G0, G1)], sem)
        c0.wait()
        c1.wait()

        def row_body(j, carry2):
            for dd in range(EMBED // 16):
                sl = pl.ds(dd * 16, 16)
                plsc.addupdate(buf.at[j, sl], pos_v[j, sl])
            return carry2

        lax.fori_loop(0, MAXLEN, row_body, 0, unroll=2)
        pltpu.sync_copy(buf, out_hbm.at[seq_base + s])
        return carry

    lax.fori_loop(0, SEQ_PER_W, seq_body, 0)


@functools.partial(jax.jit, static_argnames=())
def kernel(x, token_table, pos_table):
    run = pl.kernel(
        _sc_body,
        out_type=jax.ShapeDtypeStruct((BATCH, MAXLEN, EMBED), jnp.float32),
        mesh=plsc.VectorSubcoreMesh(core_axis_name="c", subcore_axis_name="s"),
        compiler_params=pltpu.CompilerParams(use_tc_tiling_on_sc=False),
        scratch_types=[
            pltpu.VMEM((SEQ_PER_W, MAXLEN), jnp.int32),
            pltpu.VMEM((MAXLEN, EMBED), jnp.float32),
            pltpu.VMEM((MAXLEN, EMBED), jnp.float32),
            pltpu.SemaphoreType.DMA,
        ],
    )
    return run(token_table, x.astype(jnp.int32), pos_table)
